# Initial kernel scaffold; baseline (speedup 1.0000x reference)
import jax
import jax.numpy as jnp
from jax.experimental import pallas as pl


def kernel(x, edge_index, Wl0, bl0, Wr0, g0, be0, Wl1, bl1, Wr1, g1, be1, Wl2, bl2, Wr2, g2, be2, Wh1, bh1, Wh2, bh2):
    raise NotImplementedError("write your pallas kernel here")


# speedup vs baseline: 5.2554x; 5.2554x over previous
import functools

import jax
import jax.numpy as jnp
from jax import lax
from jax.experimental import pallas as pl
from jax.experimental.pallas import tpu as pltpu
from jax.experimental.pallas import tpu_sc as plsc

N = 10000
NP = 10240
D = 128
E = 320000
NC = 2
NS = 16
NW = NC * NS
EPW = E // NW
K = 80
NCHUNK = EPW // K
RPT = NP // NS
RB = 128
CH = NP // D

_f32 = jnp.float32


def _sc_agg_body(with_counts, *refs):
    if with_counts:
        (z_hbm, src_hbm, dst_hbm, zrow_hbm,
         s_hbm, co_hbm, acc_sh, zbuf, sidx, didx, rows, cnt, gsem) = refs
    else:
        (z_hbm, src_hbm, dst_hbm, zrow_hbm,
         s_hbm, acc_sh, zbuf, sidx, didx, rows, gsem) = refs

    c = lax.axis_index("c")
    s = lax.axis_index("s")
    wid = c * NS + s

    pltpu.sync_copy(zrow_hbm, zbuf)
    for j in range(RPT // RB):
        pltpu.sync_copy(zbuf, acc_sh.at[pl.ds(s * RPT + j * RB, RB)])
    if with_counts:
        pltpu.sync_copy(zrow_hbm.at[pl.ds(0, CH)], cnt)
    plsc.subcore_barrier()

    base = wid * EPW
    ones16 = jnp.ones((16,), _f32)

    def chunk(i, carry):
        off = base + i * K
        pltpu.sync_copy(src_hbm.at[pl.ds(off, K)], sidx)
        pltpu.sync_copy(dst_hbm.at[pl.ds(off, K)], didx)
        pltpu.async_copy(z_hbm.at[sidx], rows, gsem).wait()
        pltpu.sync_copy(rows, acc_sh.at[didx], add=True)
        if with_counts:
            for t in range(K // 16):
                d = didx[pl.ds(t * 16, 16)]
                plsc.addupdate_scatter(
                    cnt, [lax.shift_right_logical(d, 7), lax.bitwise_and(d, 127)],
                    ones16)
        return carry

    lax.fori_loop(0, NCHUNK, chunk, 0)
    plsc.subcore_barrier()

    for j in range(RPT // RB):
        pltpu.sync_copy(acc_sh.at[pl.ds(s * RPT + j * RB, RB)], zbuf)
        pltpu.sync_copy(zbuf, s_hbm.at[c, pl.ds(s * RPT + j * RB, RB)])
    if with_counts:
        pltpu.sync_copy(cnt, co_hbm.at[wid])


def _make_sc_agg(with_counts):
    out_type = [jax.ShapeDtypeStruct((NC, NP, D), _f32)]
    scratch = []
    if with_counts:
        out_type.append(jax.ShapeDtypeStruct((NW, CH, D), _f32))
    scratch.append(pltpu.VMEM_SHARED((NP, D), _f32))
    scratch.append(pltpu.VMEM((RB, D), _f32))
    scratch.append(pltpu.VMEM((K,), jnp.int32))
    scratch.append(pltpu.VMEM((K,), jnp.int32))
    scratch.append(pltpu.VMEM((K, D), _f32))
    if with_counts:
        scratch.append(pltpu.VMEM((CH, D), _f32))
    scratch.append(pltpu.SemaphoreType.DMA)

    mesh = plsc.VectorSubcoreMesh(core_axis_name="c", subcore_axis_name="s")
    return pl.kernel(
        functools.partial(_sc_agg_body, with_counts),
        out_type=tuple(out_type) if len(out_type) > 1 else out_type[0],
        mesh=mesh,
        scratch_types=tuple(scratch),
        compiler_params=pltpu.CompilerParams(needs_layout_passes=False),
        name="sc_edge_agg_cnt" if with_counts else "sc_edge_agg",
    )


_sc_agg_counts = _make_sc_agg(True)
_sc_agg = _make_sc_agg(False)


def _tc_z0_body(x_ref, w_ref, o_ref):
    o_ref[...] = jnp.dot(x_ref[...], w_ref[...],
                         preferred_element_type=_f32)


_tc_z0 = pl.pallas_call(
    _tc_z0_body,
    out_shape=jax.ShapeDtypeStruct((N, D), _f32),
)


def _tc_csum_body(c_ref, o_ref):
    o_ref[...] = jnp.sum(c_ref[...], axis=0)


_tc_csum = pl.pallas_call(
    _tc_csum_body,
    out_shape=jax.ShapeDtypeStruct((CH, D), _f32),
)


def _bn_relu(u, g_ref, be_ref):
    m = jnp.mean(u, axis=0, keepdims=True)
    var = jnp.mean((u - m) ** 2, axis=0, keepdims=True)
    return jnp.maximum(
        g_ref[...] * (u - m) * lax.rsqrt(var + 1e-5) + be_ref[...], 0.0)


def _sage_update(s_ref, c_ref, h_ref, wr_ref, bl_ref):
    inv = 1.0 / jnp.maximum(c_ref[0:N], 1.0)
    return ((s_ref[0, :N, :] + s_ref[1, :N, :]) * inv + bl_ref[...]
            + jnp.dot(h_ref[...], wr_ref[...], preferred_element_type=_f32))


def _tc_combine_body(s_ref, c_ref, h_ref, wr_ref, bl_ref, g_ref, be_ref,
                     wln_ref, hn_ref, zn_ref):
    hn = _bn_relu(_sage_update(s_ref, c_ref, h_ref, wr_ref, bl_ref),
                  g_ref, be_ref)
    hn_ref[...] = hn
    zn_ref[...] = jnp.dot(hn, wln_ref[...], preferred_element_type=_f32)


_tc_combine = pl.pallas_call(
    _tc_combine_body,
    out_shape=(jax.ShapeDtypeStruct((N, D), _f32),
               jax.ShapeDtypeStruct((N, D), _f32)),
)


def _tc_final_body(s_ref, c_ref, h_ref, wr_ref, bl_ref, g_ref, be_ref,
                   wh1_ref, bh1_ref, wh2_ref, bh2_ref, out_ref):
    hn = _bn_relu(_sage_update(s_ref, c_ref, h_ref, wr_ref, bl_ref),
                  g_ref, be_ref)
    t = jnp.maximum(
        jnp.dot(hn, wh1_ref[...], preferred_element_type=_f32)
        + bh1_ref[...], 0.0)
    out_ref[...] = (jnp.dot(t, wh2_ref[...], preferred_element_type=_f32)
                    + bh2_ref[...])


_tc_final = pl.pallas_call(
    _tc_final_body,
    out_shape=jax.ShapeDtypeStruct((N, 2), _f32),
)


def kernel(x, edge_index, Wl0, bl0, Wr0, g0, be0, Wl1, bl1, Wr1, g1, be1,
           Wl2, bl2, Wr2, g2, be2, Wh1, bh1, Wh2, bh2):
    src = edge_index[0]
    dst = edge_index[1]
    zrow = jnp.zeros((RB, D), _f32)

    r2 = lambda v: v.reshape(1, -1)

    z0 = _tc_z0(x, Wl0)
    S0, C32 = _sc_agg_counts(z0, src, dst, zrow)
    C = _tc_csum(C32).reshape(NP, 1)
    h1, z1 = _tc_combine(S0, C, x, Wr0, r2(bl0), r2(g0), r2(be0), Wl1)
    S1 = _sc_agg(z1, src, dst, zrow)
    h2, z2 = _tc_combine(S1, C, h1, Wr1, r2(bl1), r2(g1), r2(be1), Wl2)
    S2 = _sc_agg(z2, src, dst, zrow)
    return _tc_final(S2, C, h2, Wr2, r2(bl2), r2(g2), r2(be2),
                     Wh1, r2(bh1), Wh2, r2(bh2))


# speedup vs baseline: 8.3657x; 1.5918x over previous
import functools

import jax
import jax.numpy as jnp
from jax import lax
from jax.experimental import pallas as pl
from jax.experimental.pallas import tpu as pltpu
from jax.experimental.pallas import tpu_sc as plsc

N = 10000
NP = 10240
D = 128
E = 320000
NC = 2
NS = 16
NW = NC * NS
EPW = E // NW
K = 80
NCHUNK = EPW // K
RPT = NP // NS
RB = 128
CH = NP // D

_f32 = jnp.float32


def _sc_agg_body(with_counts, *refs):
    if with_counts:
        (z_hbm, src_hbm, dst_hbm, zrow_hbm, s_hbm, co_hbm, acc_sh, zbuf,
         sidx0, sidx1, didx0, didx1, rows0, rows1, cnt, gsem0, gsem1) = refs
    else:
        (z_hbm, src_hbm, dst_hbm, zrow_hbm, s_hbm, acc_sh, zbuf,
         sidx0, sidx1, didx0, didx1, rows0, rows1, gsem0, gsem1) = refs
    sidxs, didxs = (sidx0, sidx1), (didx0, didx1)
    rowss, gsems = (rows0, rows1), (gsem0, gsem1)

    c = lax.axis_index("c")
    s = lax.axis_index("s")
    wid = c * NS + s

    pltpu.sync_copy(zrow_hbm, zbuf)
    for j in range(RPT // RB):
        pltpu.sync_copy(zbuf, acc_sh.at[pl.ds(s * RPT + j * RB, RB)])
    if with_counts:
        pltpu.sync_copy(zrow_hbm.at[pl.ds(0, CH)], cnt)
    plsc.subcore_barrier()

    base = wid * EPW
    ones16 = jnp.ones((16,), _f32)

    def count_scatter(didx):
        for t in range(K // 16):
            d = didx[pl.ds(t * 16, 16)]
            plsc.addupdate_scatter(
                cnt, [lax.shift_right_logical(d, 7), lax.bitwise_and(d, 127)],
                ones16)

    pltpu.sync_copy(src_hbm.at[pl.ds(base, K)], sidxs[0])
    pltpu.sync_copy(dst_hbm.at[pl.ds(base, K)], didxs[0])
    pltpu.async_copy(z_hbm.at[sidxs[0]], rowss[0], gsems[0])

    def pair(p, carry):
        for b in (0, 1):
            ci = 2 * p + b
            nxt = base + (ci + 1) * K
            pltpu.sync_copy(src_hbm.at[pl.ds(nxt, K)], sidxs[1 - b])
            pltpu.sync_copy(dst_hbm.at[pl.ds(nxt, K)], didxs[1 - b])
            pltpu.make_async_copy(z_hbm.at[sidxs[b]], rowss[b],
                                  gsems[b]).wait()
            pltpu.async_copy(z_hbm.at[sidxs[1 - b]], rowss[1 - b],
                             gsems[1 - b])
            pltpu.sync_copy(rowss[b], acc_sh.at[didxs[b]], add=True)
            if with_counts:
                count_scatter(didxs[b])
        return carry

    lax.fori_loop(0, (NCHUNK - 1) // 2, pair, 0)
    pltpu.make_async_copy(z_hbm.at[sidxs[0]], rowss[0], gsems[0]).wait()
    pltpu.sync_copy(rowss[0], acc_sh.at[didxs[0]], add=True)
    if with_counts:
        count_scatter(didxs[0])
    plsc.subcore_barrier()

    for j in range(RPT // RB):
        pltpu.sync_copy(acc_sh.at[pl.ds(s * RPT + j * RB, RB)], zbuf)
        pltpu.sync_copy(zbuf, s_hbm.at[c, pl.ds(s * RPT + j * RB, RB)])
    if with_counts:
        pltpu.sync_copy(cnt, co_hbm.at[wid])


def _make_sc_agg(with_counts):
    out_type = [jax.ShapeDtypeStruct((NC, NP, D), _f32)]
    scratch = []
    if with_counts:
        out_type.append(jax.ShapeDtypeStruct((NW, CH, D), _f32))
    scratch.append(pltpu.VMEM_SHARED((NP, D), _f32))
    scratch.append(pltpu.VMEM((RB, D), _f32))
    scratch.append(pltpu.VMEM((K,), jnp.int32))
    scratch.append(pltpu.VMEM((K,), jnp.int32))
    scratch.append(pltpu.VMEM((K,), jnp.int32))
    scratch.append(pltpu.VMEM((K,), jnp.int32))
    scratch.append(pltpu.VMEM((K, D), _f32))
    scratch.append(pltpu.VMEM((K, D), _f32))
    if with_counts:
        scratch.append(pltpu.VMEM((CH, D), _f32))
    scratch.append(pltpu.SemaphoreType.DMA)
    scratch.append(pltpu.SemaphoreType.DMA)

    mesh = plsc.VectorSubcoreMesh(core_axis_name="c", subcore_axis_name="s")
    return pl.kernel(
        functools.partial(_sc_agg_body, with_counts),
        out_type=tuple(out_type) if len(out_type) > 1 else out_type[0],
        mesh=mesh,
        scratch_types=tuple(scratch),
        compiler_params=pltpu.CompilerParams(needs_layout_passes=False),
        name="sc_edge_agg_cnt" if with_counts else "sc_edge_agg",
    )


_sc_agg_counts = _make_sc_agg(True)
_sc_agg = _make_sc_agg(False)


def _tc_z0_body(x_ref, w_ref, o_ref):
    o_ref[...] = jnp.dot(x_ref[...], w_ref[...],
                         preferred_element_type=_f32)


_tc_z0 = pl.pallas_call(
    _tc_z0_body,
    out_shape=jax.ShapeDtypeStruct((N, D), _f32),
)


def _tc_csum_body(c_ref, o_ref):
    o_ref[...] = jnp.sum(c_ref[...], axis=0)


_tc_csum = pl.pallas_call(
    _tc_csum_body,
    out_shape=jax.ShapeDtypeStruct((CH, D), _f32),
)


def _bn_relu(u, g_ref, be_ref):
    m = jnp.mean(u, axis=0, keepdims=True)
    var = jnp.mean((u - m) ** 2, axis=0, keepdims=True)
    return jnp.maximum(
        g_ref[...] * (u - m) * lax.rsqrt(var + 1e-5) + be_ref[...], 0.0)


def _sage_update(s_ref, c_ref, h_ref, wr_ref, bl_ref):
    inv = 1.0 / jnp.maximum(c_ref[0:N], 1.0)
    return ((s_ref[0, :N, :] + s_ref[1, :N, :]) * inv + bl_ref[...]
            + jnp.dot(h_ref[...], wr_ref[...], preferred_element_type=_f32))


def _tc_combine_body(s_ref, c_ref, h_ref, wr_ref, bl_ref, g_ref, be_ref,
                     wln_ref, hn_ref, zn_ref):
    hn = _bn_relu(_sage_update(s_ref, c_ref, h_ref, wr_ref, bl_ref),
                  g_ref, be_ref)
    hn_ref[...] = hn
    zn_ref[...] = jnp.dot(hn, wln_ref[...], preferred_element_type=_f32)


_tc_combine = pl.pallas_call(
    _tc_combine_body,
    out_shape=(jax.ShapeDtypeStruct((N, D), _f32),
               jax.ShapeDtypeStruct((N, D), _f32)),
)


def _tc_final_body(s_ref, c_ref, h_ref, wr_ref, bl_ref, g_ref, be_ref,
                   wh1_ref, bh1_ref, wh2_ref, bh2_ref, out_ref):
    hn = _bn_relu(_sage_update(s_ref, c_ref, h_ref, wr_ref, bl_ref),
                  g_ref, be_ref)
    t = jnp.maximum(
        jnp.dot(hn, wh1_ref[...], preferred_element_type=_f32)
        + bh1_ref[...], 0.0)
    out_ref[...] = (jnp.dot(t, wh2_ref[...], preferred_element_type=_f32)
                    + bh2_ref[...])


_tc_final = pl.pallas_call(
    _tc_final_body,
    out_shape=jax.ShapeDtypeStruct((N, 2), _f32),
)


def kernel(x, edge_index, Wl0, bl0, Wr0, g0, be0, Wl1, bl1, Wr1, g1, be1,
           Wl2, bl2, Wr2, g2, be2, Wh1, bh1, Wh2, bh2):
    src = edge_index[0]
    dst = edge_index[1]
    zrow = jnp.zeros((RB, D), _f32)

    r2 = lambda v: v.reshape(1, -1)

    z0 = _tc_z0(x, Wl0)
    S0, C32 = _sc_agg_counts(z0, src, dst, zrow)
    C = _tc_csum(C32).reshape(NP, 1)
    h1, z1 = _tc_combine(S0, C, x, Wr0, r2(bl0), r2(g0), r2(be0), Wl1)
    S1 = _sc_agg(z1, src, dst, zrow)
    h2, z2 = _tc_combine(S1, C, h1, Wr1, r2(bl1), r2(g1), r2(be1), Wl2)
    S2 = _sc_agg(z2, src, dst, zrow)
    return _tc_final(S2, C, h2, Wr2, r2(bl2), r2(g2), r2(be2),
                     Wh1, r2(bh1), Wh2, r2(bh2))
